# Initial kernel scaffold; baseline (speedup 1.0000x reference)
import functools

import jax
import jax.numpy as jnp
from jax import lax
from jax.experimental import pallas as pl
from jax.experimental.pallas import tpu as pltpu
from jax.experimental.pallas import tpu_sc as plsc

NUM_GRAPHS = 256
D = 128
CH = 128

_info = plsc.get_sparse_core_info()
NC = _info.num_cores
NS = _info.num_subcores
NW = NC * NS

N_CELLS = 320000
N_TRACKS = 160000
CELL_CHUNKS = N_CELLS // CH
TRACK_CHUNKS = N_TRACKS // CH


def _sc_partials(cells_feat, cells_ids, tracks_feat, tracks_ids):
    mesh = plsc.VectorSubcoreMesh(core_axis_name="c", subcore_axis_name="s")
    f32 = jnp.float32

    @functools.partial(
        pl.kernel,
        mesh=mesh,
        out_type=[
            jax.ShapeDtypeStruct((NC, NUM_GRAPHS, D), f32),
            jax.ShapeDtypeStruct((NC, NUM_GRAPHS, D), f32),
            jax.ShapeDtypeStruct((NC, NUM_GRAPHS, 16), f32),
            jax.ShapeDtypeStruct((NC, NUM_GRAPHS, 16), f32),
        ],
        scratch_types=[
            pltpu.VMEM((CH, D), f32),
            pltpu.VMEM((CH,), jnp.int32),
            pltpu.VMEM((CH, 16), f32),
            pltpu.VMEM((16, D), f32),
            pltpu.VMEM((16, 16), f32),
            pltpu.VMEM_SHARED((NUM_GRAPHS, D), f32),
            pltpu.VMEM_SHARED((NUM_GRAPHS, D), f32),
            pltpu.VMEM_SHARED((NUM_GRAPHS, 16), f32),
            pltpu.VMEM_SHARED((NUM_GRAPHS, 16), f32),
            pltpu.SemaphoreType.DMA,
            pltpu.SemaphoreType.DMA,
        ],
    )
    def k(cells_hbm, cids_hbm, tracks_hbm, tids_hbm,
          out_cs, out_ts, out_cc, out_tc,
          rows_v, ids_v, ones_v, zrow_v, zcnt_v,
          acc_c, acc_t, cnt_c, cnt_t, sem_r, sem_i):
        c = lax.axis_index("c")
        s = lax.axis_index("s")
        w = s * NC + c

        one16 = jnp.ones((16,), f32)
        zero16 = jnp.zeros((16,), f32)
        for i in range(16):
            for j in range(D // 16):
                zrow_v[i, pl.ds(j * 16, 16)] = zero16
            zcnt_v[i, pl.ds(0, 16)] = zero16
        for i in range(CH):
            ones_v[i, pl.ds(0, 16)] = one16

        rows_per_tile = NUM_GRAPHS // NS
        sl = pl.ds(s * rows_per_tile, rows_per_tile)
        pltpu.sync_copy(zrow_v, acc_c.at[sl])
        pltpu.sync_copy(zrow_v, acc_t.at[sl])
        pltpu.sync_copy(zcnt_v, cnt_c.at[sl])
        pltpu.sync_copy(zcnt_v, cnt_t.at[sl])
        plsc.subcore_barrier()

        def make_loop(feat_hbm, ids_hbm, acc, cnt):
            def body(g, start):
                gg = start + g
                base = pl.multiple_of(gg * CH, CH)
                cp_r = pltpu.async_copy(feat_hbm.at[pl.ds(base, CH)], rows_v, sem_r)
                cp_i = pltpu.async_copy(ids_hbm.at[pl.ds(base, CH)], ids_v, sem_i)
                cp_r.wait()
                cp_i.wait()
                pltpu.sync_copy(rows_v, acc.at[ids_v], add=True)
                pltpu.sync_copy(ones_v, cnt.at[ids_v], add=True)
                return start
            return body

        nk_c = 78 + jnp.where(w < 4, 1, 0)
        st_c = w * 78 + jnp.minimum(w, 4)
        lax.fori_loop(0, nk_c, make_loop(cells_hbm, cids_hbm, acc_c, cnt_c), st_c)

        nk_t = 39 + jnp.where(w < 2, 1, 0)
        st_t = w * 39 + jnp.minimum(w, 2)
        lax.fori_loop(0, nk_t, make_loop(tracks_hbm, tids_hbm, acc_t, cnt_t), st_t)

        plsc.subcore_barrier()

        pltpu.sync_copy(acc_c.at[sl], out_cs.at[c, sl])
        pltpu.sync_copy(acc_t.at[sl], out_ts.at[c, sl])
        pltpu.sync_copy(cnt_c.at[sl], out_cc.at[c, sl])
        pltpu.sync_copy(cnt_t.at[sl], out_tc.at[c, sl])

    return k(cells_feat, cells_ids, tracks_feat, tracks_ids)


def _tc_head_body(cs_ref, ts_ref, cc_ref, tc_ref, w_ref, b_ref, o_ref):
    cs = cs_ref[0] + cs_ref[1]
    ts = ts_ref[0] + ts_ref[1]
    cc = cc_ref[0] + cc_ref[1]
    tc = tc_ref[0] + tc_ref[1]
    mc = cs / jnp.maximum(cc[:, 0:1], 1.0)
    mt = ts / jnp.maximum(tc[:, 0:1], 1.0)
    ag = jnp.concatenate([mc, mt], axis=1)
    o_ref[...] = (
        jnp.dot(ag, w_ref[...], preferred_element_type=jnp.float32) + b_ref[...]
    )


def _tc_head(cs, ts, cc, tc, W, b):
    return pl.pallas_call(
        _tc_head_body,
        out_shape=jax.ShapeDtypeStruct((NUM_GRAPHS, 2), jnp.float32),
    )(cs, ts, cc, tc, W, b.reshape(1, 2))


def kernel(cells_feat, tracks_feat, W, b, cells_segment_ids, tracks_segment_ids):
    cids = cells_segment_ids.astype(jnp.int32)
    tids = tracks_segment_ids.astype(jnp.int32)
    cs, ts, cc, tc = _sc_partials(cells_feat, cids, tracks_feat, tids)
    return _tc_head(cs, ts, cc, tc, W, b)


# speedup vs baseline: 4.3523x; 4.3523x over previous
import functools

import jax
import jax.numpy as jnp
from jax import lax
from jax.experimental import pallas as pl
from jax.experimental.pallas import tpu as pltpu
from jax.experimental.pallas import tpu_sc as plsc

NUM_GRAPHS = 256
D = 128
CH = 128
L = 16

_info = plsc.get_sparse_core_info()
NC = _info.num_cores
NS = _info.num_subcores
NW = NC * NS

N_CELLS = 320000
N_TRACKS = 160000
CELL_CHUNKS = N_CELLS // CH
TRACK_CHUNKS = N_TRACKS // CH


def _sc_partials(cells_feat, cells_ids, tracks_feat, tracks_ids):
    mesh = plsc.VectorSubcoreMesh(core_axis_name="c", subcore_axis_name="s")
    f32 = jnp.float32

    @functools.partial(
        pl.kernel,
        mesh=mesh,
        out_type=[
            jax.ShapeDtypeStruct((NC, NUM_GRAPHS, D), f32),
            jax.ShapeDtypeStruct((NC, NUM_GRAPHS, D), f32),
            jax.ShapeDtypeStruct((NC, NUM_GRAPHS, D), f32),
            jax.ShapeDtypeStruct((NC, NUM_GRAPHS, D), f32),
        ],
        scratch_types=[
            pltpu.VMEM((CH, D), f32),
            pltpu.VMEM((CH,), jnp.int32),
            pltpu.VMEM((16, D), f32),
            pltpu.VMEM((CH, D), f32),
            pltpu.VMEM_SHARED((NUM_GRAPHS, D), f32),
            pltpu.VMEM_SHARED((NUM_GRAPHS, D), f32),
            pltpu.VMEM_SHARED((NUM_GRAPHS, D), f32),
            pltpu.VMEM_SHARED((NUM_GRAPHS, D), f32),
            pltpu.SemaphoreType.DMA,
            pltpu.SemaphoreType.DMA,
        ],
    )
    def k(cells_hbm, cids_hbm, tracks_hbm, tids_hbm,
          out_cs, out_ts, out_cc, out_tc,
          rows_v, ids_v, zrow_v, ones_v,
          acc_c, acc_t, cnt_c, cnt_t, sem_r, sem_i):
        c = lax.axis_index("c")
        s = lax.axis_index("s")
        w = s * NC + c

        zero16 = jnp.zeros((L,), f32)
        one16 = jnp.ones((L,), f32)
        for i in range(16):
            for j in range(D // L):
                zrow_v[i, pl.ds(j * L, L)] = zero16

        def obody(i, _):
            for j in range(D // L):
                ones_v[i, pl.ds(j * L, L)] = one16
            return 0
        lax.fori_loop(0, CH, obody, 0)

        rows_per_tile = NUM_GRAPHS // NS
        sl = pl.ds(s * rows_per_tile, rows_per_tile)
        pltpu.sync_copy(zrow_v, acc_c.at[sl])
        pltpu.sync_copy(zrow_v, acc_t.at[sl])
        pltpu.sync_copy(zrow_v, cnt_c.at[sl])
        pltpu.sync_copy(zrow_v, cnt_t.at[sl])
        plsc.subcore_barrier()

        def make_loop(feat_hbm, ids_hbm, acc, cnt):
            def body(g, start):
                gg = start + g
                base = pl.multiple_of(gg * CH, CH)
                cp_r = pltpu.async_copy(feat_hbm.at[pl.ds(base, CH)], rows_v, sem_r)
                cp_i = pltpu.async_copy(ids_hbm.at[pl.ds(base, CH)], ids_v, sem_i)
                cp_i.wait()
                pltpu.sync_copy(ones_v, cnt.at[ids_v], add=True)
                cp_r.wait()
                pltpu.sync_copy(rows_v, acc.at[ids_v], add=True)
                return start
            return body

        nk_c = 78 + jnp.where(w < 4, 1, 0)
        st_c = w * 78 + jnp.minimum(w, 4)
        lax.fori_loop(0, nk_c, make_loop(cells_hbm, cids_hbm, acc_c, cnt_c), st_c)

        nk_t = 39 + jnp.where(w < 2, 1, 0)
        st_t = w * 39 + jnp.minimum(w, 2)
        lax.fori_loop(0, nk_t, make_loop(tracks_hbm, tids_hbm, acc_t, cnt_t), st_t)

        plsc.subcore_barrier()

        pltpu.sync_copy(acc_c.at[sl], out_cs.at[c, sl])
        pltpu.sync_copy(acc_t.at[sl], out_ts.at[c, sl])
        pltpu.sync_copy(cnt_c.at[sl], out_cc.at[c, sl])
        pltpu.sync_copy(cnt_t.at[sl], out_tc.at[c, sl])

    return k(cells_feat, cells_ids, tracks_feat, tracks_ids)


def _tc_head_body(cs_ref, ts_ref, cc_ref, tc_ref, w_ref, b_ref, o_ref):
    cs = cs_ref[0] + cs_ref[1]
    ts = ts_ref[0] + ts_ref[1]
    cc = cc_ref[0, :, 0:1] + cc_ref[1, :, 0:1]
    tc = tc_ref[0, :, 0:1] + tc_ref[1, :, 0:1]
    mc = cs / jnp.maximum(cc, 1.0)
    mt = ts / jnp.maximum(tc, 1.0)
    ag = jnp.concatenate([mc, mt], axis=1)
    o_ref[...] = (
        jnp.dot(ag, w_ref[...], preferred_element_type=jnp.float32) + b_ref[...]
    )


def _tc_head(cs, ts, cc, tc, W, b):
    return pl.pallas_call(
        _tc_head_body,
        out_shape=jax.ShapeDtypeStruct((NUM_GRAPHS, 2), jnp.float32),
    )(cs, ts, cc, tc, W, b.reshape(1, 2))


def kernel(cells_feat, tracks_feat, W, b, cells_segment_ids, tracks_segment_ids):
    cids = cells_segment_ids.astype(jnp.int32)
    tids = tracks_segment_ids.astype(jnp.int32)
    cs, ts, cc, tc = _sc_partials(cells_feat, cids, tracks_feat, tids)
    return _tc_head(cs, ts, cc, tc, W, b)
